# baseline (speedup 1.0000x reference)
import functools

import jax
import jax.numpy as jnp
from jax import lax
from jax.experimental import pallas as pl
from jax.experimental.pallas import tpu as pltpu
from jax.experimental.pallas import tpu_sc as plsc

NC = 2
NS = 16
NW = NC * NS
L = 16
K = 128


def _mesh():
    return plsc.VectorSubcoreMesh(
        core_axis_name="c", subcore_axis_name="s", num_cores=NC, num_subcores=NS
    )


def _pick_chunk(epw):
    for k in range(128, 7, -8):
        if epw % k == 0:
            return k
    raise ValueError(f"no valid chunk size for {epw} edges per worker")


def _sc_deg(dst_r, n_nodes):
    nw, chunks, k = dst_r.shape
    kv = k // L

    @functools.partial(
        pl.kernel,
        out_type=jax.ShapeDtypeStruct((nw, n_nodes), jnp.float32),
        mesh=_mesh(),
        scratch_types=[
            pltpu.VMEM((chunks, k), jnp.int32),
            pltpu.VMEM((n_nodes,), jnp.float32),
        ],
        compiler_params=pltpu.CompilerParams(
            needs_layout_passes=False, use_tc_tiling_on_sc=False),
    )
    def deg_kernel(dst_hbm, out_hbm, dst_v, hist_v):
        c = lax.axis_index("c")
        s = lax.axis_index("s")
        wid = s * NC + c

        zeros = jnp.zeros((L,), jnp.float32)

        def zero_body(i, _):
            hist_v[pl.ds(i * L, L)] = zeros
            return ()

        lax.fori_loop(0, n_nodes // L, zero_body, (), unroll=8)

        pltpu.sync_copy(dst_hbm.at[wid], dst_v)

        ones = jnp.ones((L,), jnp.float32)

        def chunk_body(i, _):
            for j in range(kv):
                idx = dst_v[i, pl.ds(j * L, L)]
                plsc.addupdate_scatter(hist_v, [idx], ones)
            return ()

        lax.fori_loop(0, chunks, chunk_body, ())

        pltpu.sync_copy(hist_v, out_hbm.at[wid])

    return deg_kernel(dst_r)


def _tc_prep(deg_partial, w):
    n, d = w.shape

    def prep_kernel(degp_ref, w_ref, dinv_ref, wp_ref):
        deg = jnp.sum(degp_ref[...], axis=0) + 1.0
        dinv = lax.rsqrt(deg)
        dinv_ref[...] = dinv
        wp_ref[...] = w_ref[...] * dinv[:, None]

    return pl.pallas_call(
        prep_kernel,
        out_shape=[
            jax.ShapeDtypeStruct((n,), jnp.float32),
            jax.ShapeDtypeStruct((n, d), jnp.float32),
        ],
    )(deg_partial, w)


def _sc_msg(wp_r, src_r, dst_r, dinv, x, b):
    nc, nrows, dh = wp_r.shape
    n = nrows - 8
    d = nc * dh
    ns, ept = src_r.shape
    bsz = x.shape[0]
    bpt = bsz // NS
    hb = bpt // 2
    assert ept % L == 0 and n % L == 0 and bsz % L == 0 and hb <= K
    nv = ept // L
    NB = 4
    pk = NB * K
    cap = ((ept + pk - 1) // pk + 1) * pk
    npc = (n // NS) // 8 * 8
    tail = n - NS * npc
    zrows = 104
    assert npc % zrows == 0 and tail % 8 == 0 and tail <= zrows
    nzc = npc // zrows

    @functools.partial(
        pl.kernel,
        out_type=[
            jax.ShapeDtypeStruct((bsz, d), jnp.float32),
            jax.ShapeDtypeStruct((nc, n, dh), jnp.float32),
        ],
        mesh=_mesh(),
        scratch_types=[
            pltpu.VMEM((cap,), jnp.int32),
            pltpu.VMEM((cap,), jnp.int32),
            pltpu.VMEM((NB, K, dh), jnp.float32),
            pltpu.VMEM((bsz,), jnp.int32),
            pltpu.VMEM((n,), jnp.float32),
            pltpu.VMEM((bpt,), jnp.float32),
            pltpu.VMEM((dh,), jnp.float32),
            pltpu.VMEM_SHARED((n, dh), jnp.float32),
            pltpu.SemaphoreType.DMA,
            pltpu.SemaphoreType.DMA,
            pltpu.SemaphoreType.DMA,
            pltpu.SemaphoreType.DMA,
        ],
        compiler_params=pltpu.CompilerParams(
            needs_layout_passes=False, use_tc_tiling_on_sc=False),
    )
    def msg_kernel(wp_hbm, src_hbm, dst_hbm, dinv_hbm, x_hbm, b_hbm,
                   out2_hbm, out_hbm,
                   src_v, dst_v, rows_v, xfull, mtab, dv, bv, acc_sh,
                   sem0, sem1, sem2, sem3):
        c = lax.axis_index("c")
        s = lax.axis_index("s")
        zero_v = rows_v.at[0]

        zeros = jnp.zeros((L,), jnp.float32)

        def zero_body(i, _):
            r = i // (dh // L)
            t = i % (dh // L)
            zero_v[r, pl.ds(t * L, L)] = zeros
            return ()

        lax.fori_loop(0, zrows * (dh // L), zero_body, (), unroll=8)
        row0 = s * npc
        for z in range(nzc):
            pltpu.sync_copy(zero_v.at[pl.ds(0, zrows), :],
                            acc_sh.at[pl.ds(row0 + z * zrows, zrows), :])
        if tail:
            @pl.when(s == NS - 1)
            def _():
                pltpu.sync_copy(
                    zero_v.at[pl.ds(0, tail), :],
                    acc_sh.at[pl.ds(NS * npc, tail), :],
                )

        pltpu.sync_copy(src_hbm.at[s], src_v.at[pl.ds(0, ept)])
        pltpu.sync_copy(dst_hbm.at[s], dst_v.at[pl.ds(0, ept)])
        pltpu.sync_copy(x_hbm, xfull)

        def mz_body(i, _):
            mtab[pl.ds(i * L, L)] = zeros
            return ()

        lax.fori_loop(0, n // L, mz_body, (), unroll=8)
        ones16 = jnp.ones((L,), jnp.float32)

        def mb_body(i, _):
            plsc.store_scatter(mtab, [xfull[pl.ds(i * L, L)]], ones16)
            return ()

        lax.fori_loop(0, bsz // L, mb_body, ())

        def cp_body(v, off):
            s16 = src_v[pl.ds(v * L, L)]
            d16 = dst_v[pl.ds(v * L, L)]
            m = plsc.load_gather(mtab, [d16]) > 0.5
            plsc.store_compressed(src_v.at[pl.ds(off, L)], s16, mask=m)
            plsc.store_compressed(dst_v.at[pl.ds(off, L)], d16, mask=m)
            return off + plsc.all_reduce_population_count(m)[0]

        nkept = lax.fori_loop(0, nv, cp_body, jnp.int32(0))

        npad = (jnp.maximum(nkept, 1) + pk - 1) // pk * pk
        dsrc = jnp.full((L,), n, jnp.int32)
        ddst = jnp.zeros((L,), jnp.int32)
        tmask = jnp.ones((L,), jnp.bool_)

        def pad_body(i, _):
            o = nkept + i * L
            plsc.store_compressed(src_v.at[pl.ds(o, L)], dsrc, mask=tmask)
            plsc.store_compressed(dst_v.at[pl.ds(o, L)], ddst, mask=tmask)
            return ()

        lax.fori_loop(0, (npad - nkept + L - 1) // L, pad_body, ())
        nch = npad // K

        plsc.subcore_barrier()

        sems = (sem0, sem1, sem2, sem3)

        def gather(i, bb):
            pltpu.async_copy(
                wp_hbm.at[c].at[src_v.at[pl.ds(i * K, K)]],
                rows_v.at[bb], sems[bb])

        for i in range(NB - 1):
            gather(i, i)

        def chunk_body(g, _):
            for bb in range(NB):
                i = g * NB + bb
                pltpu.make_async_copy(
                    wp_hbm.at[c].at[src_v.at[pl.ds(i * K, K)]],
                    rows_v.at[bb], sems[bb],
                ).wait()
                h = i + NB - 1
                bh = (bb + NB - 1) % NB

                @pl.when(h < nch)
                def _():
                    gather(h, bh)

                pltpu.sync_copy(rows_v.at[bb],
                                acc_sh.at[dst_v.at[pl.ds(i * K, K)]],
                                add=True)
            return ()

        lax.fori_loop(0, nch // NB, chunk_body, ())

        plsc.subcore_barrier()
        pltpu.sync_copy(
            acc_sh.at[pl.ds(row0, npc), :], out_hbm.at[c, pl.ds(row0, npc), :]
        )
        if tail:
            @pl.when(s == NS - 1)
            def _():
                pltpu.sync_copy(
                    acc_sh.at[pl.ds(NS * npc, tail), :],
                    out_hbm.at[c, pl.ds(NS * npc, tail), :],
                )

        plsc.subcore_barrier()
        pltpu.sync_copy(dinv_hbm, mtab)
        pltpu.sync_copy(b_hbm.at[pl.ds(c * dh, dh)], bv)
        arow = rows_v.at[0]
        wrow = rows_v.at[1]
        base = s * bpt
        tph = dh // L

        for half in range(2):
            o = base + half * hb
            ca = pltpu.async_copy(
                out_hbm.at[c].at[xfull.at[pl.ds(o, hb)]],
                arow.at[pl.ds(0, hb), :], sem0)
            cw = pltpu.async_copy(
                wp_hbm.at[c].at[xfull.at[pl.ds(o, hb)]],
                wrow.at[pl.ds(0, hb), :], sem1)
            ca.wait()
            cw.wait()

            def dv_body(j, _):
                idx = xfull[pl.ds(o + j * L, L)]
                dv[pl.ds(j * L, L)] = plsc.load_gather(mtab, [idx])
                return ()

            lax.fori_loop(0, hb // L, dv_body, ())

            def row_body(jo, _):
                dvec = dv[pl.ds(jo * L, L)]
                for r in range(L):
                    j = jo * L + r
                    dscale = dvec[r]
                    for t in range(tph):
                        sl = pl.ds(t * L, L)
                        arow[j, sl] = \
                            (arow[j, sl] + wrow[j, sl]) * dscale + bv[sl]
                return ()

            lax.fori_loop(0, hb // L, row_body, ())

            pltpu.sync_copy(
                arow.at[pl.ds(0, hb), :],
                out2_hbm.at[pl.ds(o, hb), pl.ds(c * dh, dh)])

    return msg_kernel(wp_r, src_r, dst_r, dinv, x, b)


def kernel(x, edge_index, W, b):
    n, d = W.shape
    e = edge_index.shape[1]
    dh = d // NC

    epw = e // NW
    kd = _pick_chunk(epw)
    dst_d = edge_index[1].astype(jnp.int32).reshape(NW, epw // kd, kd)

    ept = e // NS
    src_r = edge_index[0].astype(jnp.int32).reshape(NS, ept)
    dst_r = edge_index[1].astype(jnp.int32).reshape(NS, ept)
    x32 = x.astype(jnp.int32)

    deg_partial = _sc_deg(dst_d, n)
    dinv, wp = _tc_prep(deg_partial, W)
    wp_r = wp.reshape(n, NC, dh).transpose(1, 0, 2)
    wp_r = jnp.concatenate(
        [wp_r, jnp.zeros((NC, 8, dh), jnp.float32)], axis=1)
    out, _ = _sc_msg(wp_r, src_r, dst_r, dinv, x32, b)
    return out


# speedup vs baseline: 1.2626x; 1.2626x over previous
import functools

import jax
import jax.numpy as jnp
from jax import lax
from jax.experimental import pallas as pl
from jax.experimental.pallas import tpu as pltpu
from jax.experimental.pallas import tpu_sc as plsc

NC = 2
NS = 16
NW = NC * NS
L = 16
K = 128


def _mesh():
    return plsc.VectorSubcoreMesh(
        core_axis_name="c", subcore_axis_name="s", num_cores=NC, num_subcores=NS
    )


def _pick_chunk(epw):
    for k in range(128, 7, -8):
        if epw % k == 0:
            return k
    raise ValueError(f"no valid chunk size for {epw} edges per worker")


def _sc_deg(dst_r, n_nodes):
    nw, chunks, k = dst_r.shape
    kv = k // L

    @functools.partial(
        pl.kernel,
        out_type=jax.ShapeDtypeStruct((nw, n_nodes), jnp.float32),
        mesh=_mesh(),
        scratch_types=[
            pltpu.VMEM((chunks, k), jnp.int32),
            pltpu.VMEM((n_nodes,), jnp.float32),
        ],
        compiler_params=pltpu.CompilerParams(
            needs_layout_passes=False, use_tc_tiling_on_sc=False),
    )
    def deg_kernel(dst_hbm, out_hbm, dst_v, hist_v):
        c = lax.axis_index("c")
        s = lax.axis_index("s")
        wid = s * NC + c

        zeros = jnp.zeros((L,), jnp.float32)

        def zero_body(i, _):
            hist_v[pl.ds(i * L, L)] = zeros
            return ()

        lax.fori_loop(0, n_nodes // L, zero_body, (), unroll=8)

        pltpu.sync_copy(dst_hbm.at[wid], dst_v)

        ones = jnp.ones((L,), jnp.float32)

        def chunk_body(i, _):
            for j in range(kv):
                idx = dst_v[i, pl.ds(j * L, L)]
                plsc.addupdate_scatter(hist_v, [idx], ones)
            return ()

        lax.fori_loop(0, chunks, chunk_body, ())

        pltpu.sync_copy(hist_v, out_hbm.at[wid])

    return deg_kernel(dst_r)


def _tc_prep(deg_partial, w):
    n, d = w.shape

    def prep_kernel(degp_ref, w_ref, dinv_ref, wp_ref):
        deg = jnp.sum(degp_ref[...], axis=0) + 1.0
        dinv = lax.rsqrt(deg)
        dinv_ref[...] = dinv
        wp_ref[...] = w_ref[...] * dinv[:, None]

    return pl.pallas_call(
        prep_kernel,
        out_shape=[
            jax.ShapeDtypeStruct((n,), jnp.float32),
            jax.ShapeDtypeStruct((n, d), jnp.float32),
        ],
    )(deg_partial, w)


def _sc_msg(wp_r, src_r, dst_r, dinv, x, b):
    nc, nrows, dh = wp_r.shape
    n = nrows - 8
    d = nc * dh
    ns, ept = src_r.shape
    bsz = x.shape[0]
    bpt = bsz // NS
    hb = bpt // 2
    assert ept % L == 0 and n % L == 0 and bsz % L == 0 and hb <= K
    nv = ept // L
    NB = 2
    pk = NB * K
    cap = ((ept + pk - 1) // pk + 1) * pk
    npc = (n // NS) // 8 * 8
    tail = n - NS * npc
    zrows = 104
    assert npc % zrows == 0 and tail % 8 == 0 and tail <= zrows
    nzc = npc // zrows

    @functools.partial(
        pl.kernel,
        out_type=[
            jax.ShapeDtypeStruct((bsz, d), jnp.float32),
            jax.ShapeDtypeStruct((nc, n, dh), jnp.float32),
        ],
        mesh=_mesh(),
        scratch_types=[
            pltpu.VMEM((cap,), jnp.int32),
            pltpu.VMEM((cap,), jnp.int32),
            pltpu.VMEM((NB, K, dh), jnp.float32),
            pltpu.VMEM((bsz,), jnp.int32),
            pltpu.VMEM((n,), jnp.float32),
            pltpu.VMEM((bpt,), jnp.float32),
            pltpu.VMEM((dh,), jnp.float32),
            pltpu.VMEM_SHARED((n, dh), jnp.float32),
            pltpu.SemaphoreType.DMA,
            pltpu.SemaphoreType.DMA,
        ],
        compiler_params=pltpu.CompilerParams(
            needs_layout_passes=False, use_tc_tiling_on_sc=False),
    )
    def msg_kernel(wp_hbm, src_hbm, dst_hbm, dinv_hbm, x_hbm, b_hbm,
                   out2_hbm, out_hbm,
                   src_v, dst_v, rows_v, xfull, mtab, dv, bv, acc_sh,
                   sem0, sem1):
        c = lax.axis_index("c")
        s = lax.axis_index("s")
        zero_v = rows_v.at[0]

        zeros = jnp.zeros((L,), jnp.float32)

        def zero_body(i, _):
            r = i // (dh // L)
            t = i % (dh // L)
            zero_v[r, pl.ds(t * L, L)] = zeros
            return ()

        lax.fori_loop(0, zrows * (dh // L), zero_body, (), unroll=8)
        row0 = s * npc
        for z in range(nzc):
            pltpu.sync_copy(zero_v.at[pl.ds(0, zrows), :],
                            acc_sh.at[pl.ds(row0 + z * zrows, zrows), :])
        if tail:
            @pl.when(s == NS - 1)
            def _():
                pltpu.sync_copy(
                    zero_v.at[pl.ds(0, tail), :],
                    acc_sh.at[pl.ds(NS * npc, tail), :],
                )

        pltpu.sync_copy(src_hbm.at[s], src_v.at[pl.ds(0, ept)])
        pltpu.sync_copy(dst_hbm.at[s], dst_v.at[pl.ds(0, ept)])
        pltpu.sync_copy(x_hbm, xfull)

        def mz_body(i, _):
            mtab[pl.ds(i * L, L)] = zeros
            return ()

        lax.fori_loop(0, n // L, mz_body, (), unroll=8)
        ones16 = jnp.ones((L,), jnp.float32)

        def mb_body(i, _):
            plsc.store_scatter(mtab, [xfull[pl.ds(i * L, L)]], ones16)
            return ()

        lax.fori_loop(0, bsz // L, mb_body, ())

        def cp_body(v, off):
            s16 = src_v[pl.ds(v * L, L)]
            d16 = dst_v[pl.ds(v * L, L)]
            m = plsc.load_gather(mtab, [d16]) > 0.5
            plsc.store_compressed(src_v.at[pl.ds(off, L)], s16, mask=m)
            plsc.store_compressed(dst_v.at[pl.ds(off, L)], d16, mask=m)
            return off + plsc.all_reduce_population_count(m)[0]

        nkept = lax.fori_loop(0, nv, cp_body, jnp.int32(0))

        npad = (jnp.maximum(nkept, 1) + pk - 1) // pk * pk
        dsrc = jnp.full((L,), n, jnp.int32)
        ddst = jnp.zeros((L,), jnp.int32)
        tmask = jnp.ones((L,), jnp.bool_)

        def pad_body(i, _):
            o = nkept + i * L
            plsc.store_compressed(src_v.at[pl.ds(o, L)], dsrc, mask=tmask)
            plsc.store_compressed(dst_v.at[pl.ds(o, L)], ddst, mask=tmask)
            return ()

        lax.fori_loop(0, (npad - nkept + L - 1) // L, pad_body, ())
        nch = npad // K

        plsc.subcore_barrier()

        sems = (sem0, sem1)
        pltpu.async_copy(
            wp_hbm.at[c].at[src_v.at[pl.ds(0, K)]], rows_v.at[0], sem0)

        def chunk_body(g, _):
            for bb in range(2):
                i = g * 2 + bb
                pltpu.make_async_copy(
                    wp_hbm.at[c].at[src_v.at[pl.ds(i * K, K)]],
                    rows_v.at[bb], sems[bb],
                ).wait()

                @pl.when(i + 1 < nch)
                def _():
                    pltpu.async_copy(
                        wp_hbm.at[c].at[src_v.at[pl.ds((i + 1) * K, K)]],
                        rows_v.at[1 - bb],
                        sems[1 - bb],
                    )

                pltpu.sync_copy(rows_v.at[bb],
                                acc_sh.at[dst_v.at[pl.ds(i * K, K)]],
                                add=True)
            return ()

        lax.fori_loop(0, nch // 2, chunk_body, ())

        plsc.subcore_barrier()
        pltpu.sync_copy(
            acc_sh.at[pl.ds(row0, npc), :], out_hbm.at[c, pl.ds(row0, npc), :]
        )
        if tail:
            @pl.when(s == NS - 1)
            def _():
                pltpu.sync_copy(
                    acc_sh.at[pl.ds(NS * npc, tail), :],
                    out_hbm.at[c, pl.ds(NS * npc, tail), :],
                )

        plsc.subcore_barrier()
        pltpu.sync_copy(dinv_hbm, mtab)
        pltpu.sync_copy(b_hbm.at[pl.ds(c * dh, dh)], bv)
        arow = rows_v.at[0]
        wrow = rows_v.at[1]
        base = s * bpt
        tph = dh // L

        for half in range(2):
            o = base + half * hb
            ca = pltpu.async_copy(
                out_hbm.at[c].at[xfull.at[pl.ds(o, hb)]],
                arow.at[pl.ds(0, hb), :], sem0)
            cw = pltpu.async_copy(
                wp_hbm.at[c].at[xfull.at[pl.ds(o, hb)]],
                wrow.at[pl.ds(0, hb), :], sem1)
            ca.wait()
            cw.wait()

            def dv_body(j, _):
                idx = xfull[pl.ds(o + j * L, L)]
                dv[pl.ds(j * L, L)] = plsc.load_gather(mtab, [idx])
                return ()

            lax.fori_loop(0, hb // L, dv_body, ())

            def row_body(jo, _):
                dvec = dv[pl.ds(jo * L, L)]
                for r in range(L):
                    j = jo * L + r
                    dscale = dvec[r]
                    for t in range(tph):
                        sl = pl.ds(t * L, L)
                        arow[j, sl] = \
                            (arow[j, sl] + wrow[j, sl]) * dscale + bv[sl]
                return ()

            lax.fori_loop(0, hb // L, row_body, ())

            pltpu.sync_copy(
                arow.at[pl.ds(0, hb), :],
                out2_hbm.at[pl.ds(o, hb), pl.ds(c * dh, dh)])

    return msg_kernel(wp_r, src_r, dst_r, dinv, x, b)


def kernel(x, edge_index, W, b):
    n, d = W.shape
    e = edge_index.shape[1]
    dh = d // NC

    epw = e // NW
    kd = _pick_chunk(epw)
    dst_d = edge_index[1].astype(jnp.int32).reshape(NW, epw // kd, kd)

    ept = e // NS
    src_r = edge_index[0].astype(jnp.int32).reshape(NS, ept)
    dst_r = edge_index[1].astype(jnp.int32).reshape(NS, ept)
    x32 = x.astype(jnp.int32)

    deg_partial = _sc_deg(dst_d, n)
    dinv, wp = _tc_prep(deg_partial, W)
    wp_r = wp.reshape(n, NC, dh).transpose(1, 0, 2)
    wp_r = jnp.concatenate(
        [wp_r, jnp.zeros((NC, 8, dh), jnp.float32)], axis=1)
    out, _ = _sc_msg(wp_r, src_r, dst_r, dinv, x32, b)
    return out


# speedup vs baseline: 1.2640x; 1.0011x over previous
import functools

import jax
import jax.numpy as jnp
from jax import lax
from jax.experimental import pallas as pl
from jax.experimental.pallas import tpu as pltpu
from jax.experimental.pallas import tpu_sc as plsc

NC = 2
NS = 16
NW = NC * NS
L = 16
K = 128


def _mesh():
    return plsc.VectorSubcoreMesh(
        core_axis_name="c", subcore_axis_name="s", num_cores=NC, num_subcores=NS
    )


def _pick_chunk(epw):
    for k in range(128, 7, -8):
        if epw % k == 0:
            return k
    raise ValueError(f"no valid chunk size for {epw} edges per worker")


def _sc_deg(dst_r, n_nodes):
    nw, chunks, k = dst_r.shape
    kv = k // L

    @functools.partial(
        pl.kernel,
        out_type=jax.ShapeDtypeStruct((nw, n_nodes), jnp.float32),
        mesh=_mesh(),
        scratch_types=[
            pltpu.VMEM((chunks, k), jnp.int32),
            pltpu.VMEM((n_nodes,), jnp.float32),
        ],
        compiler_params=pltpu.CompilerParams(
            needs_layout_passes=False, use_tc_tiling_on_sc=False),
    )
    def deg_kernel(dst_hbm, out_hbm, dst_v, hist_v):
        c = lax.axis_index("c")
        s = lax.axis_index("s")
        wid = s * NC + c

        zeros = jnp.zeros((L,), jnp.float32)

        def zero_body(i, _):
            hist_v[pl.ds(i * L, L)] = zeros
            return ()

        lax.fori_loop(0, n_nodes // L, zero_body, (), unroll=8)

        pltpu.sync_copy(dst_hbm.at[wid], dst_v)

        ones = jnp.ones((L,), jnp.float32)

        def chunk_body(i, _):
            for j in range(kv):
                idx = dst_v[i, pl.ds(j * L, L)]
                plsc.addupdate_scatter(hist_v, [idx], ones)
            return ()

        lax.fori_loop(0, chunks, chunk_body, ())

        pltpu.sync_copy(hist_v, out_hbm.at[wid])

    return deg_kernel(dst_r)


def _tc_prep(deg_partial, w):
    n, d = w.shape

    def prep_kernel(degp_ref, w_ref, dinv_ref, wp_ref):
        deg = jnp.sum(degp_ref[...], axis=0) + 1.0
        dinv = lax.rsqrt(deg)
        dinv_ref[...] = dinv
        wp_ref[...] = w_ref[...] * dinv[:, None]

    return pl.pallas_call(
        prep_kernel,
        out_shape=[
            jax.ShapeDtypeStruct((n,), jnp.float32),
            jax.ShapeDtypeStruct((n, d), jnp.float32),
        ],
    )(deg_partial, w)


def _sc_msg(wp_r, src_r, dst_r, dinv, x, b):
    nc, nrows, dh = wp_r.shape
    n = nrows - 8
    d = nc * dh
    ns, ept = src_r.shape
    bsz = x.shape[0]
    bpt = bsz // NS
    hb = bpt // 2
    assert ept % L == 0 and n % L == 0 and bsz % L == 0 and hb <= K
    nv = ept // L
    NB = 2
    pk = NB * K
    cap = ((ept + pk - 1) // pk + 1) * pk
    npc = (n // NS) // 8 * 8
    tail = n - NS * npc
    zrows = 104
    assert npc % zrows == 0 and tail % 8 == 0 and tail <= zrows
    nzc = npc // zrows

    @functools.partial(
        pl.kernel,
        out_type=[
            jax.ShapeDtypeStruct((bsz, d), jnp.float32),
            jax.ShapeDtypeStruct((nc, n, dh), jnp.float32),
        ],
        mesh=_mesh(),
        scratch_types=[
            pltpu.VMEM((cap,), jnp.int32),
            pltpu.VMEM((cap,), jnp.int32),
            pltpu.VMEM((NB, K, dh), jnp.float32),
            pltpu.VMEM((bsz,), jnp.int32),
            pltpu.VMEM((n,), jnp.float32),
            pltpu.VMEM((bpt,), jnp.float32),
            pltpu.VMEM((dh,), jnp.float32),
            pltpu.VMEM_SHARED((n, dh), jnp.float32),
            pltpu.SemaphoreType.DMA,
            pltpu.SemaphoreType.DMA,
        ],
        compiler_params=pltpu.CompilerParams(
            needs_layout_passes=False, use_tc_tiling_on_sc=False),
    )
    def msg_kernel(wp_hbm, src_hbm, dst_hbm, dinv_hbm, x_hbm, b_hbm,
                   out2_hbm, out_hbm,
                   src_v, dst_v, rows_v, xfull, mtab, dv, bv, acc_sh,
                   sem0, sem1):
        c = lax.axis_index("c")
        s = lax.axis_index("s")
        zero_v = rows_v.at[0]

        zeros = jnp.zeros((L,), jnp.float32)

        def zero_body(i, _):
            r = i // (dh // L)
            t = i % (dh // L)
            zero_v[r, pl.ds(t * L, L)] = zeros
            return ()

        lax.fori_loop(0, zrows * (dh // L), zero_body, (), unroll=8)
        row0 = s * npc
        for z in range(nzc):
            pltpu.sync_copy(zero_v.at[pl.ds(0, zrows), :],
                            acc_sh.at[pl.ds(row0 + z * zrows, zrows), :])
        if tail:
            @pl.when(s == NS - 1)
            def _():
                pltpu.sync_copy(
                    zero_v.at[pl.ds(0, tail), :],
                    acc_sh.at[pl.ds(NS * npc, tail), :],
                )

        pltpu.sync_copy(src_hbm.at[s], src_v.at[pl.ds(0, ept)])
        pltpu.sync_copy(dst_hbm.at[s], dst_v.at[pl.ds(0, ept)])
        pltpu.sync_copy(x_hbm, xfull)

        def mz_body(i, _):
            mtab[pl.ds(i * L, L)] = zeros
            return ()

        lax.fori_loop(0, n // L, mz_body, (), unroll=8)
        ones16 = jnp.ones((L,), jnp.float32)

        def mb_body(i, _):
            plsc.store_scatter(mtab, [xfull[pl.ds(i * L, L)]], ones16)
            return ()

        lax.fori_loop(0, bsz // L, mb_body, (), unroll=4)

        def cp_body(v, off):
            s16 = src_v[pl.ds(v * L, L)]
            d16 = dst_v[pl.ds(v * L, L)]
            m = plsc.load_gather(mtab, [d16]) > 0.5
            plsc.store_compressed(src_v.at[pl.ds(off, L)], s16, mask=m)
            plsc.store_compressed(dst_v.at[pl.ds(off, L)], d16, mask=m)
            return off + plsc.all_reduce_population_count(m)[0]

        nkept = lax.fori_loop(0, nv, cp_body, jnp.int32(0), unroll=2)

        npad = (jnp.maximum(nkept, 1) + pk - 1) // pk * pk
        dsrc = jnp.full((L,), n, jnp.int32)
        ddst = jnp.zeros((L,), jnp.int32)
        tmask = jnp.ones((L,), jnp.bool_)

        def pad_body(i, _):
            o = nkept + i * L
            plsc.store_compressed(src_v.at[pl.ds(o, L)], dsrc, mask=tmask)
            plsc.store_compressed(dst_v.at[pl.ds(o, L)], ddst, mask=tmask)
            return ()

        lax.fori_loop(0, (npad - nkept + L - 1) // L, pad_body, ())
        nch = npad // K

        plsc.subcore_barrier()

        sems = (sem0, sem1)
        pltpu.async_copy(
            wp_hbm.at[c].at[src_v.at[pl.ds(0, K)]], rows_v.at[0], sem0)

        def chunk_body(g, _):
            for bb in range(2):
                i = g * 2 + bb
                pltpu.make_async_copy(
                    wp_hbm.at[c].at[src_v.at[pl.ds(i * K, K)]],
                    rows_v.at[bb], sems[bb],
                ).wait()

                @pl.when(i + 1 < nch)
                def _():
                    pltpu.async_copy(
                        wp_hbm.at[c].at[src_v.at[pl.ds((i + 1) * K, K)]],
                        rows_v.at[1 - bb],
                        sems[1 - bb],
                    )

                pltpu.sync_copy(rows_v.at[bb],
                                acc_sh.at[dst_v.at[pl.ds(i * K, K)]],
                                add=True)
            return ()

        lax.fori_loop(0, nch // 2, chunk_body, ())

        plsc.subcore_barrier()
        pltpu.sync_copy(
            acc_sh.at[pl.ds(row0, npc), :], out_hbm.at[c, pl.ds(row0, npc), :]
        )
        if tail:
            @pl.when(s == NS - 1)
            def _():
                pltpu.sync_copy(
                    acc_sh.at[pl.ds(NS * npc, tail), :],
                    out_hbm.at[c, pl.ds(NS * npc, tail), :],
                )

        plsc.subcore_barrier()
        pltpu.sync_copy(dinv_hbm, mtab)
        pltpu.sync_copy(b_hbm.at[pl.ds(c * dh, dh)], bv)
        arow = rows_v.at[0]
        wrow = rows_v.at[1]
        base = s * bpt
        tph = dh // L

        for half in range(2):
            o = base + half * hb
            ca = pltpu.async_copy(
                out_hbm.at[c].at[xfull.at[pl.ds(o, hb)]],
                arow.at[pl.ds(0, hb), :], sem0)
            cw = pltpu.async_copy(
                wp_hbm.at[c].at[xfull.at[pl.ds(o, hb)]],
                wrow.at[pl.ds(0, hb), :], sem1)
            ca.wait()
            cw.wait()

            def dv_body(j, _):
                idx = xfull[pl.ds(o + j * L, L)]
                dv[pl.ds(j * L, L)] = plsc.load_gather(mtab, [idx])
                return ()

            lax.fori_loop(0, hb // L, dv_body, ())

            def row_body(jo, _):
                dvec = dv[pl.ds(jo * L, L)]
                for r in range(L):
                    j = jo * L + r
                    dscale = dvec[r]
                    for t in range(tph):
                        sl = pl.ds(t * L, L)
                        arow[j, sl] = \
                            (arow[j, sl] + wrow[j, sl]) * dscale + bv[sl]
                return ()

            lax.fori_loop(0, hb // L, row_body, ())

            pltpu.sync_copy(
                arow.at[pl.ds(0, hb), :],
                out2_hbm.at[pl.ds(o, hb), pl.ds(c * dh, dh)])

    return msg_kernel(wp_r, src_r, dst_r, dinv, x, b)


def kernel(x, edge_index, W, b):
    n, d = W.shape
    e = edge_index.shape[1]
    dh = d // NC

    epw = e // NW
    kd = _pick_chunk(epw)
    dst_d = edge_index[1].astype(jnp.int32).reshape(NW, epw // kd, kd)

    ept = e // NS
    src_r = edge_index[0].astype(jnp.int32).reshape(NS, ept)
    dst_r = edge_index[1].astype(jnp.int32).reshape(NS, ept)
    x32 = x.astype(jnp.int32)

    deg_partial = _sc_deg(dst_d, n)
    dinv, wp = _tc_prep(deg_partial, W)
    wp_r = wp.reshape(n, NC, dh).transpose(1, 0, 2)
    wp_r = jnp.concatenate(
        [wp_r, jnp.zeros((NC, 8, dh), jnp.float32)], axis=1)
    out, _ = _sc_msg(wp_r, src_r, dst_r, dinv, x32, b)
    return out
